# baseline (speedup 1.0000x reference)
import functools

import jax
import jax.numpy as jnp
from jax import lax
from jax.experimental import pallas as pl
from jax.experimental.pallas import tpu as pltpu
from jax.experimental.pallas import tpu_sc as plsc

VOCAB = 100000
D = 1024
BATCH = 4
SEQ = 2048
NC, NS = 2, 16
NW = NC * NS
PP = SEQ // NW
SP = 16
NSTAGE = PP // SP
CH = NSTAGE * BATCH
NBUF = 6
LANES = 16

_mesh = plsc.VectorSubcoreMesh(core_axis_name="c", subcore_axis_name="s")


@functools.partial(
    pl.kernel,
    mesh=_mesh,
    out_type=jax.ShapeDtypeStruct((BATCH, SEQ, D), jnp.float32),
    scratch_types=(
        [pltpu.VMEM((BATCH * PP,), jnp.int32)]
        + [pltpu.VMEM((SP, D), jnp.float32) for _ in range(NBUF)]
        + [pltpu.VMEM((SP, D), jnp.float32)]
        + [pltpu.SemaphoreType.DMA for _ in range(NBUF + NBUF + 1 + 1)]
    ),
)
def _embed(x_hbm, tok_hbm, pos_hbm, out_hbm, idx_v, *rest):
    toks = rest[:NBUF]
    pos_v = rest[NBUF]
    gsems = rest[NBUF + 1:2 * NBUF + 1]
    wsems = rest[2 * NBUF + 1:3 * NBUF + 1]
    psem = rest[3 * NBUF + 1]
    isem = rest[3 * NBUF + 2]

    wid = lax.axis_index("s") * NC + lax.axis_index("c")
    p_base = wid * PP

    h_idx = [
        pltpu.async_copy(
            x_hbm.at[b, pl.ds(p_base, PP)],
            idx_v.at[pl.ds(b * PP, PP)],
            isem,
        )
        for b in range(BATCH)
    ]
    idx_ready = [False] * BATCH

    def load_pos(t):
        return pltpu.async_copy(
            pos_hbm.at[pl.ds(p_base + t * SP, SP)], pos_v, psem
        )

    def gather(c):
        t, b = divmod(c, BATCH)
        if not idx_ready[b]:
            h_idx[b].wait()
            idx_ready[b] = True
        off = b * PP + t * SP
        return pltpu.async_copy(
            tok_hbm.at[idx_v.at[pl.ds(off, SP)]], toks[c % NBUF], gsems[c % NBUF]
        )

    h_pos = [None] * NSTAGE
    h_pos[0] = load_pos(0)
    h_g = [None] * CH
    h_w = [None] * CH
    for c in range(4):
        h_g[c] = gather(c)

    for c in range(CH):
        t, b = divmod(c, BATCH)
        if b == 0:
            h_pos[t].wait()
        h_g[c].wait()
        if b % 2 == 1:
            bufa = toks[(c - 1) % NBUF]
            bufb = toks[c % NBUF]

            def _row(i, carry):
                for k in range(D // LANES):
                    sl = pl.ds(k * LANES, LANES)
                    v = pos_v[i, sl]
                    plsc.addupdate(bufa.at[i, sl], v)
                    plsc.addupdate(bufb.at[i, sl], v)
                return carry

            lax.fori_loop(0, SP, _row, 0)
            row0 = p_base + t * SP
            h_w[c - 1] = pltpu.async_copy(
                bufa, out_hbm.at[b - 1, pl.ds(row0, SP)], wsems[(c - 1) % NBUF]
            )
            h_w[c] = pltpu.async_copy(
                bufb, out_hbm.at[b, pl.ds(row0, SP)], wsems[c % NBUF]
            )
            if b == BATCH - 1 and t + 1 < NSTAGE:
                h_pos[t + 1] = load_pos(t + 1)
            if c + 3 < CH:
                if c >= 3:
                    h_w[c - 3].wait()
                    h_w[c - 2].wait()
                h_g[c + 3] = gather(c + 3)
                h_g[c + 4] = gather(c + 4)

    for c in range(10, CH):
        h_w[c].wait()


def kernel(x, token_table, position_table):
    return _embed(x.astype(jnp.int32), token_table, position_table)


# speedup vs baseline: 1.0546x; 1.0546x over previous
import functools

import jax
import jax.numpy as jnp
from jax import lax
from jax.experimental import pallas as pl
from jax.experimental.pallas import tpu as pltpu
from jax.experimental.pallas import tpu_sc as plsc

VOCAB = 100000
D = 1024
BATCH = 4
SEQ = 2048
NC, NS = 2, 16
NW = NC * NS
PP = SEQ // NW
SP = 16
NSTAGE = PP // SP
CH = NSTAGE * BATCH
NBUF = 5
LANES = 16

_mesh = plsc.VectorSubcoreMesh(core_axis_name="c", subcore_axis_name="s")


@functools.partial(
    pl.kernel,
    mesh=_mesh,
    out_type=jax.ShapeDtypeStruct((BATCH, SEQ, D), jnp.float32),
    scratch_types=(
        [pltpu.VMEM((BATCH * PP,), jnp.int32)]
        + [pltpu.VMEM((SP, D), jnp.float32) for _ in range(NBUF)]
        + [pltpu.VMEM((SP, D), jnp.float32) for _ in range(2)]
        + [pltpu.SemaphoreType.DMA for _ in range(NBUF + NBUF + 2 + 1)]
    ),
)
def _embed(x_hbm, tok_hbm, pos_hbm, out_hbm, idx_v, *rest):
    toks = rest[:NBUF]
    poss = rest[NBUF:NBUF + 2]
    gsems = rest[NBUF + 2:2 * NBUF + 2]
    wsems = rest[2 * NBUF + 2:3 * NBUF + 2]
    psems = rest[3 * NBUF + 2:3 * NBUF + 4]
    isem = rest[3 * NBUF + 4]

    wid = lax.axis_index("s") * NC + lax.axis_index("c")
    p_base = wid * PP

    h_idx = [
        pltpu.async_copy(
            x_hbm.at[b, pl.ds(p_base, PP)],
            idx_v.at[pl.ds(b * PP, PP)],
            isem,
        )
        for b in range(BATCH)
    ]
    idx_ready = [False] * BATCH

    def load_pos(t):
        return pltpu.async_copy(
            pos_hbm.at[pl.ds(p_base + t * SP, SP)], poss[t % 2], psems[t % 2]
        )

    h_pos = [None] * NSTAGE
    for t in range(min(2, NSTAGE)):
        h_pos[t] = load_pos(t)

    def gather(c):
        t, b = divmod(c, BATCH)
        if not idx_ready[b]:
            h_idx[b].wait()
            idx_ready[b] = True
        off = b * PP + t * SP
        return pltpu.async_copy(
            tok_hbm.at[idx_v.at[pl.ds(off, SP)]], toks[c % NBUF], gsems[c % NBUF]
        )

    h_g = [None] * CH
    h_w = [None] * CH
    for c in range(NBUF - 1):
        h_g[c] = gather(c)

    for c in range(CH):
        t, b = divmod(c, BATCH)
        if b == 0:
            h_pos[t].wait()
        h_g[c].wait()
        buf = toks[c % NBUF]
        pbuf = poss[t % 2]

        def _row(i, carry):
            for k in range(D // LANES):
                sl = pl.ds(k * LANES, LANES)
                plsc.addupdate(buf.at[i, sl], pbuf[i, sl])
            return carry

        lax.fori_loop(0, SP, _row, 0)
        h_w[c] = pltpu.async_copy(
            buf, out_hbm.at[b, pl.ds(p_base + t * SP, SP)], wsems[c % NBUF]
        )
        if b == BATCH - 1 and t + 2 < NSTAGE:
            h_pos[t + 2] = load_pos(t + 2)
        if c + NBUF - 1 < CH:
            if c >= 1:
                h_w[c - 1].wait()
            h_g[c + NBUF - 1] = gather(c + NBUF - 1)

    for c in range(CH - NBUF, CH):
        h_w[c].wait()


def kernel(x, token_table, position_table):
    return _embed(x.astype(jnp.int32), token_table, position_table)
